# baseline (speedup 1.0000x reference)
import functools

import jax
import jax.numpy as jnp
from jax import lax
from jax.experimental import pallas as pl
from jax.experimental.pallas import tpu as pltpu
from jax.experimental.pallas import tpu_sc as plsc

_D = 64
_B = 16384
_NC = 2
_NS = 16
_NW = _NC * _NS
_BPW = _B // _NW
_CH = 128
_NCH = _BPW // _CH
_L = 16
_G = _D // _L
_NQ = 5


def _sc_partials(idx_hbm, table_hbm, out_hbm, idx_v, x_v, x1_v, x2_v,
                 acc_v, sem):
    wid = lax.axis_index("s") * _NC + lax.axis_index("c")
    pltpu.sync_copy(idx_hbm.at[wid], idx_v)

    rows = (x_v, x1_v, x2_v)
    copies = []
    for t in range(3):
        for j in range(_NCH):
            copies.append(pltpu.async_copy(
                table_hbm.at[idx_v.at[t * _NCH + j]],
                rows[t].at[pl.ds(j * _CH, _CH)], sem))
    for c in copies:
        c.wait()

    zeros = jnp.zeros((_L,), jnp.float32)

    def body(b, accs):
        out = []
        for g in range(_G):
            x = x_v[b, pl.ds(g * _L, _L)]
            x1 = x1_v[b, pl.ds(g * _L, _L)]
            x2 = x2_v[b, pl.ds(g * _L, _L)]
            a = accs[g * _NQ:(g + 1) * _NQ]
            out.extend((a[0] + x * x1, a[1] + x * x2, a[2] + x * x,
                        a[3] + x1 * x1, a[4] + x2 * x2))
        return tuple(out)

    accs = lax.fori_loop(0, _BPW, body, (zeros,) * (_NQ * _G))
    for g in range(_G):
        for q in range(_NQ):
            acc_v[q, pl.ds(g * _L, _L)] = accs[g * _NQ + q]
    pltpu.sync_copy(acc_v, out_hbm.at[wid])


def _tc_combine(p_ref, o_ref):
    s = jnp.sum(p_ref[...], axis=0)
    num1 = s[0:1, :]
    num2 = s[1:2, :]
    nxx = jnp.sqrt(s[2:3, :])
    n11 = jnp.sqrt(s[3:4, :])
    n22 = jnp.sqrt(s[4:5, :])
    one = num1 / jnp.maximum(nxx * n11, 1e-6)
    two = num2 / jnp.maximum(nxx * n22, 1e-6)
    o_ref[...] = two - one


@jax.jit
def kernel(inputs_, embeddings):
    idx = inputs_.reshape(3, _NW, _NCH, _CH).transpose(1, 0, 2, 3)
    idx = idx.reshape(_NW, 3 * _NCH, _CH)

    mesh = plsc.VectorSubcoreMesh(core_axis_name="c", subcore_axis_name="s")
    partials = pl.kernel(
        _sc_partials,
        mesh=mesh,
        compiler_params=pltpu.CompilerParams(use_tc_tiling_on_sc=False),
        out_type=jax.ShapeDtypeStruct((_NW, _NQ, _D), jnp.float32),
        scratch_types=[
            pltpu.VMEM((3 * _NCH, _CH), jnp.int32),
            pltpu.VMEM((_BPW, _D), jnp.float32),
            pltpu.VMEM((_BPW, _D), jnp.float32),
            pltpu.VMEM((_BPW, _D), jnp.float32),
            pltpu.VMEM((_NQ, _D), jnp.float32),
            pltpu.SemaphoreType.DMA,
        ],
    )(idx, embeddings)

    out = pl.pallas_call(
        _tc_combine,
        out_shape=jax.ShapeDtypeStruct((1, _D), jnp.float32),
    )(partials)
    return out.reshape(_D)
